# baseline (speedup 1.0000x reference)
import jax
import jax.numpy as jnp
from jax.experimental import pallas as pl
from jax.experimental.pallas import tpu as pltpu

_N = 50000
_C = 512
_IC = 128
_TN = 2000
_NT = _N // _TN
_NCACHE = 20
_G = 8
_GS = 16


def _fused_body(xl_ref, xr_ref, w1t_ref, b1_ref, g1_ref, be1_ref, wd_ref,
                wb_ref, g2_ref, be2_ref, waggt_ref, bagg_ref, o_ref,
                acc_ref, yh_ref, att_ref, xc_ref):
    p = pl.program_id(0)
    i = pl.program_id(1)
    _H = _C // 2

    @pl.when(p == 0)
    def _pass1():
        xlb = xl_ref[...].astype(jnp.bfloat16)
        xrb = xr_ref[...].astype(jnp.bfloat16)

        @pl.when(i < _NCACHE)
        def _stash():
            xc_ref[i, :, 0:_H] = xlb
            xc_ref[i, :, _H:_C] = xrb

        w1t = w1t_ref[...]
        y = jnp.dot(xlb, w1t[0:_H, :], preferred_element_type=jnp.float32) \
            + jnp.dot(xrb, w1t[_H:_C, :],
                      preferred_element_type=jnp.float32) + b1_ref[...]
        s = jnp.sum(y, axis=0, keepdims=True)
        s2 = jnp.sum(y * y, axis=0, keepdims=True)

        @pl.when(i == 0)
        def _init():
            acc_ref[0:1, :] = s
            acc_ref[1:2, :] = s2
            yh_ref[...] = y[0:_IC, :]

        @pl.when(i > 0)
        def _accum():
            acc_ref[0:1, :] = acc_ref[0:1, :] + s
            acc_ref[1:2, :] = acc_ref[1:2, :] + s2

        @pl.when(i == _NT - 1)
        def _finish():
            n = jnp.float32(_N)
            m = acc_ref[0:1, :] / n
            var = acc_ref[1:2, :] / n - m * m
            h = (yh_ref[...] - m) * jax.lax.rsqrt(var + 1e-5) * g1_ref[...] \
                + be1_ref[...]
            h = h * jax.nn.sigmoid(h)
            pr = jax.lax.broadcasted_iota(jnp.int32, (_G, _IC), 0)
            pc = jax.lax.broadcasted_iota(jnp.int32, (_G, _IC), 1)
            pool = jnp.where(pc // _GS == pr, 1.0 / _GS, 0.0)
            xs = jnp.dot(pool, h, preferred_element_type=jnp.float32)
            u = jnp.dot(xs, wd_ref[...], preferred_element_type=jnp.float32)
            v = jnp.dot(xs, wb_ref[...], preferred_element_type=jnp.float32)
            er = jax.lax.broadcasted_iota(jnp.int32, (_G * _G, _G), 0)
            ec = jax.lax.broadcasted_iota(jnp.int32, (_G * _G, _G), 1)
            rep = (er // _G == ec).astype(jnp.float32)
            til = (er % _G == ec).astype(jnp.float32)
            e_feat = jnp.dot(rep, u, preferred_element_type=jnp.float32) \
                + jnp.dot(til, v, preferred_element_type=jnp.float32)
            me = jnp.mean(e_feat, axis=0, keepdims=True)
            ve = jnp.mean(e_feat * e_feat, axis=0, keepdims=True) - me * me
            eh = (e_feat - me) * jax.lax.rsqrt(ve + 1e-5) * g2_ref[...] \
                + be2_ref[...]
            eh = eh * jax.nn.sigmoid(eh)
            sr = jax.lax.broadcasted_iota(jnp.int32, (_G, _G * _G), 0)
            sc = jax.lax.broadcasted_iota(jnp.int32, (_G, _G * _G), 1)
            seg = (sc // _G == sr).astype(jnp.float32)
            x_out = jnp.dot(seg, eh, preferred_element_type=jnp.float32)
            att = jax.nn.sigmoid(
                jnp.dot(x_out, waggt_ref[...],
                        preferred_element_type=jnp.float32) + bagg_ref[...])
            att_ref[...] = jnp.mean(att, axis=0, keepdims=True)

    @pl.when(p == 1)
    def _pass2():
        @pl.when(i < _NCACHE)
        def _from_cache():
            o_ref[...] = xc_ref[i].astype(jnp.float32) * att_ref[...]

        @pl.when(i >= _NCACHE)
        def _from_hbm():
            att = att_ref[...]
            o_ref[:, 0:_H] = xl_ref[...] * att[:, 0:_H]
            o_ref[:, _H:_C] = xr_ref[...] * att[:, _H:_C]


def kernel(x, W1, b1, gamma1, beta1, W_ec, gamma2, beta2, W_agg, b_agg):
    w1t = W1.T.astype(jnp.bfloat16)
    a = W_ec[:, :_IC]
    b = W_ec[:, _IC:]
    wd = (a - b).T
    wb = b.T
    waggt = W_agg.T
    full = lambda shape: pl.BlockSpec(shape, lambda p, i: (0, 0))

    out = pl.pallas_call(
        _fused_body,
        grid=(2, _NT),
        in_specs=[
            pl.BlockSpec((_TN, _C // 2),
                         lambda p, i: (jnp.where(p == 0, i,
                                                 jnp.maximum(i, _NCACHE)), 0)),
            pl.BlockSpec((_TN, _C // 2),
                         lambda p, i: (jnp.where(p == 0, i,
                                                 jnp.maximum(i, _NCACHE)), 1)),
            full((_C, _IC)),
            full((1, _IC)),
            full((1, _IC)),
            full((1, _IC)),
            full((_IC, _IC)),
            full((_IC, _IC)),
            full((1, _IC)),
            full((1, _IC)),
            full((_IC, _C)),
            full((1, _C)),
        ],
        out_specs=pl.BlockSpec((_TN, _C), lambda p, i: (i * p, 0)),
        out_shape=jax.ShapeDtypeStruct((_N, _C), jnp.float32),
        scratch_shapes=[
            pltpu.VMEM((2, _IC), jnp.float32),
            pltpu.VMEM((_IC, _IC), jnp.float32),
            pltpu.VMEM((1, _C), jnp.float32),
            pltpu.VMEM((_NCACHE, _TN, _C), jnp.bfloat16),
        ],
        compiler_params=pltpu.CompilerParams(
            dimension_semantics=("arbitrary", "arbitrary"),
            vmem_limit_bytes=60 * 1024 * 1024),
    )(x, x, w1t, b1.reshape(1, _IC), gamma1.reshape(1, _IC),
      beta1.reshape(1, _IC), wd, wb, gamma2.reshape(1, _IC),
      beta2.reshape(1, _IC), waggt, b_agg.reshape(1, _C))
    return out


# speedup vs baseline: 1.0171x; 1.0171x over previous
import jax
import jax.numpy as jnp
from jax.experimental import pallas as pl
from jax.experimental.pallas import tpu as pltpu

_N = 50000
_C = 512
_IC = 128
_TN = 2000
_NT = _N // _TN
_NCACHE = 21
_G = 8
_GS = 16


def _fused_body(x_ref, w1t_ref, b1_ref, g1_ref, be1_ref, wd_ref, wb_ref,
                g2_ref, be2_ref, waggt_ref, bagg_ref, o_ref,
                acc_ref, yh_ref, att_ref, xc_ref):
    p = pl.program_id(0)
    i = pl.program_id(1)

    @pl.when(p == 0)
    def _pass1():
        xb = x_ref[...].astype(jnp.bfloat16)

        @pl.when(i < _NCACHE)
        def _stash():
            xc_ref[i] = xb

        y = jnp.dot(xb, w1t_ref[...], preferred_element_type=jnp.float32)
        s = jnp.sum(y, axis=0, keepdims=True)
        s2 = jnp.sum(y * y, axis=0, keepdims=True)

        @pl.when(i == 0)
        def _init():
            acc_ref[0:1, :] = s
            acc_ref[1:2, :] = s2
            yh_ref[...] = y[0:_IC, :]

        @pl.when(i > 0)
        def _accum():
            acc_ref[0:1, :] = acc_ref[0:1, :] + s
            acc_ref[1:2, :] = acc_ref[1:2, :] + s2

        @pl.when(i == _NT - 1)
        def _finish():
            n = jnp.float32(_N)
            b1 = b1_ref[...]
            s_tot = acc_ref[0:1, :]
            s2_tot = acc_ref[1:2, :] + 2.0 * b1 * s_tot + n * b1 * b1
            m = s_tot / n + b1
            var = s2_tot / n - m * m
            h = (yh_ref[...] + b1 - m) * jax.lax.rsqrt(var + 1e-5) \
                * g1_ref[...] + be1_ref[...]
            h = h * jax.nn.sigmoid(h)
            pr = jax.lax.broadcasted_iota(jnp.int32, (_G, _IC), 0)
            pc = jax.lax.broadcasted_iota(jnp.int32, (_G, _IC), 1)
            pool = jnp.where(pc // _GS == pr, 1.0 / _GS, 0.0)
            xs = jnp.dot(pool, h, preferred_element_type=jnp.float32)
            u = jnp.dot(xs, wd_ref[...], preferred_element_type=jnp.float32)
            v = jnp.dot(xs, wb_ref[...], preferred_element_type=jnp.float32)
            er = jax.lax.broadcasted_iota(jnp.int32, (_G * _G, _G), 0)
            ec = jax.lax.broadcasted_iota(jnp.int32, (_G * _G, _G), 1)
            rep = (er // _G == ec).astype(jnp.float32)
            til = (er % _G == ec).astype(jnp.float32)
            e_feat = jnp.dot(rep, u, preferred_element_type=jnp.float32) \
                + jnp.dot(til, v, preferred_element_type=jnp.float32)
            me = jnp.mean(e_feat, axis=0, keepdims=True)
            ve = jnp.mean(e_feat * e_feat, axis=0, keepdims=True) - me * me
            eh = (e_feat - me) * jax.lax.rsqrt(ve + 1e-5) * g2_ref[...] \
                + be2_ref[...]
            eh = eh * jax.nn.sigmoid(eh)
            sr = jax.lax.broadcasted_iota(jnp.int32, (_G, _G * _G), 0)
            sc = jax.lax.broadcasted_iota(jnp.int32, (_G, _G * _G), 1)
            seg = (sc // _G == sr).astype(jnp.float32)
            x_out = jnp.dot(seg, eh, preferred_element_type=jnp.float32)
            att = jax.nn.sigmoid(
                jnp.dot(x_out, waggt_ref[...],
                        preferred_element_type=jnp.float32) + bagg_ref[...])
            att_ref[...] = jnp.mean(att, axis=0, keepdims=True)

    @pl.when(p == 1)
    def _pass2():
        @pl.when(i < _NCACHE)
        def _from_cache():
            ab = att_ref[...].astype(jnp.bfloat16)
            o_ref[...] = (xc_ref[i] * ab).astype(jnp.float32)

        @pl.when(i >= _NCACHE)
        def _from_hbm():
            o_ref[...] = x_ref[...] * att_ref[...]


def kernel(x, W1, b1, gamma1, beta1, W_ec, gamma2, beta2, W_agg, b_agg):
    w1t = W1.T.astype(jnp.bfloat16)
    a = W_ec[:, :_IC]
    b = W_ec[:, _IC:]
    wd = (a - b).T
    wb = b.T
    waggt = W_agg.T
    full = lambda shape: pl.BlockSpec(shape, lambda p, i: (0, 0))

    out = pl.pallas_call(
        _fused_body,
        grid=(2, _NT),
        in_specs=[
            pl.BlockSpec((_TN, _C),
                         lambda p, i: (jnp.where(p == 0, i,
                                                 jnp.maximum(i, _NCACHE)), 0)),
            full((_C, _IC)),
            full((1, _IC)),
            full((1, _IC)),
            full((1, _IC)),
            full((_IC, _IC)),
            full((_IC, _IC)),
            full((1, _IC)),
            full((1, _IC)),
            full((_IC, _C)),
            full((1, _C)),
        ],
        out_specs=pl.BlockSpec((_TN, _C), lambda p, i: (i * p, 0)),
        out_shape=jax.ShapeDtypeStruct((_N, _C), jnp.float32),
        scratch_shapes=[
            pltpu.VMEM((2, _IC), jnp.float32),
            pltpu.VMEM((_IC, _IC), jnp.float32),
            pltpu.VMEM((1, _C), jnp.float32),
            pltpu.VMEM((_NCACHE, _TN, _C), jnp.bfloat16),
        ],
        compiler_params=pltpu.CompilerParams(
            dimension_semantics=("arbitrary", "arbitrary"),
            vmem_limit_bytes=62 * 1024 * 1024),
    )(x, w1t, b1.reshape(1, _IC), gamma1.reshape(1, _IC),
      beta1.reshape(1, _IC), wd, wb, gamma2.reshape(1, _IC),
      beta2.reshape(1, _IC), waggt, b_agg.reshape(1, _C))
    return out


# speedup vs baseline: 1.0184x; 1.0013x over previous
import jax
import jax.numpy as jnp
from jax.experimental import pallas as pl
from jax.experimental.pallas import tpu as pltpu

_N = 50000
_C = 512
_IC = 128
_TN = 2000
_NT = _N // _TN
_NCACHE = 21
_G = 8
_GS = 16


def _fused_body(x_ref, w1t_ref, b1_ref, g1_ref, be1_ref, wd_ref, wb_ref,
                g2_ref, be2_ref, waggt_ref, bagg_ref, o_ref,
                acc_ref, yh_ref, att_ref, xc_ref):
    p = pl.program_id(0)
    i = pl.program_id(1)

    @pl.when(p == 0)
    def _pass1():
        xb = x_ref[...].astype(jnp.bfloat16)

        @pl.when(i < _NCACHE)
        def _stash():
            xc_ref[i] = xb

        y = jnp.dot(xb, w1t_ref[...], preferred_element_type=jnp.float32)
        s = jnp.sum(y, axis=0, keepdims=True)
        s2 = jnp.sum(y * y, axis=0, keepdims=True)

        @pl.when(i == 0)
        def _init():
            acc_ref[0:1, :] = s
            acc_ref[1:2, :] = s2
            yh_ref[...] = y[0:_IC, :]

        @pl.when(i > 0)
        def _accum():
            acc_ref[0:1, :] = acc_ref[0:1, :] + s
            acc_ref[1:2, :] = acc_ref[1:2, :] + s2

        @pl.when(i == _NT - 1)
        def _finish():
            n = jnp.float32(_N)
            b1 = b1_ref[...]
            s_tot = acc_ref[0:1, :]
            s2_tot = acc_ref[1:2, :] + 2.0 * b1 * s_tot + n * b1 * b1
            m = s_tot / n + b1
            var = s2_tot / n - m * m
            h = (yh_ref[...] + b1 - m) * jax.lax.rsqrt(var + 1e-5) \
                * g1_ref[...] + be1_ref[...]
            h = h * jax.nn.sigmoid(h)
            pr = jax.lax.broadcasted_iota(jnp.int32, (_G, _IC), 0)
            pc = jax.lax.broadcasted_iota(jnp.int32, (_G, _IC), 1)
            pool = jnp.where(pc // _GS == pr, 1.0 / _GS, 0.0)
            xs = jnp.dot(pool, h, preferred_element_type=jnp.float32)
            u = jnp.dot(xs, wd_ref[...], preferred_element_type=jnp.float32)
            v = jnp.dot(xs, wb_ref[...], preferred_element_type=jnp.float32)
            er = jax.lax.broadcasted_iota(jnp.int32, (_G * _G, _G), 0)
            ec = jax.lax.broadcasted_iota(jnp.int32, (_G * _G, _G), 1)
            rep = (er // _G == ec).astype(jnp.float32)
            til = (er % _G == ec).astype(jnp.float32)
            e_feat = jnp.dot(rep, u, preferred_element_type=jnp.float32) \
                + jnp.dot(til, v, preferred_element_type=jnp.float32)
            me = jnp.mean(e_feat, axis=0, keepdims=True)
            ve = jnp.mean(e_feat * e_feat, axis=0, keepdims=True) - me * me
            eh = (e_feat - me) * jax.lax.rsqrt(ve + 1e-5) * g2_ref[...] \
                + be2_ref[...]
            eh = eh * jax.nn.sigmoid(eh)
            sr = jax.lax.broadcasted_iota(jnp.int32, (_G, _G * _G), 0)
            sc = jax.lax.broadcasted_iota(jnp.int32, (_G, _G * _G), 1)
            seg = (sc // _G == sr).astype(jnp.float32)
            x_out = jnp.dot(seg, eh, preferred_element_type=jnp.float32)
            att = jax.nn.sigmoid(
                jnp.dot(x_out, waggt_ref[...],
                        preferred_element_type=jnp.float32) + bagg_ref[...])
            att_ref[...] = jnp.mean(att, axis=0, keepdims=True)

    @pl.when(p == 1)
    def _pass2():
        @pl.when(i < _NCACHE)
        def _from_cache():
            o_ref[...] = xc_ref[i].astype(jnp.float32) * att_ref[...]

        @pl.when(i >= _NCACHE)
        def _from_hbm():
            o_ref[...] = x_ref[...] * att_ref[...]


def kernel(x, W1, b1, gamma1, beta1, W_ec, gamma2, beta2, W_agg, b_agg):
    w1t = W1.T.astype(jnp.bfloat16)
    a = W_ec[:, :_IC]
    b = W_ec[:, _IC:]
    wd = (a - b).T
    wb = b.T
    waggt = W_agg.T
    full = lambda shape: pl.BlockSpec(shape, lambda p, i: (0, 0))

    out = pl.pallas_call(
        _fused_body,
        grid=(2, _NT),
        in_specs=[
            pl.BlockSpec((_TN, _C),
                         lambda p, i: (jnp.where(p == 0, i,
                                                 jnp.maximum(i, _NCACHE)), 0)),
            full((_C, _IC)),
            full((1, _IC)),
            full((1, _IC)),
            full((1, _IC)),
            full((_IC, _IC)),
            full((_IC, _IC)),
            full((1, _IC)),
            full((1, _IC)),
            full((_IC, _C)),
            full((1, _C)),
        ],
        out_specs=pl.BlockSpec((_TN, _C), lambda p, i: (i * p, 0)),
        out_shape=jax.ShapeDtypeStruct((_N, _C), jnp.float32),
        scratch_shapes=[
            pltpu.VMEM((2, _IC), jnp.float32),
            pltpu.VMEM((_IC, _IC), jnp.float32),
            pltpu.VMEM((1, _C), jnp.float32),
            pltpu.VMEM((_NCACHE, _TN, _C), jnp.bfloat16),
        ],
        compiler_params=pltpu.CompilerParams(
            dimension_semantics=("arbitrary", "arbitrary"),
            vmem_limit_bytes=62 * 1024 * 1024),
    )(x, w1t, b1.reshape(1, _IC), gamma1.reshape(1, _IC),
      beta1.reshape(1, _IC), wd, wb, gamma2.reshape(1, _IC),
      beta2.reshape(1, _IC), waggt, b_agg.reshape(1, _C))
    return out
